# baseline (speedup 1.0000x reference)
import functools

import jax
import jax.numpy as jnp
from jax import lax
from jax.experimental import pallas as pl
from jax.experimental.pallas import tpu as pltpu
from jax.experimental.pallas import tpu_sc as plsc

N_NODES = 10000
N_EDGES = 320000
D_FEAT = 128
HIDDEN = 64
N_CLASSES = 40
DP1 = 72
DP2 = 56

NC = 2
NS = 16
NW = NC * NS
EB = 80
EK = N_EDGES // NW // EB

CHUNK = 624
TAIL = N_NODES - (NS - 1) * CHUNK


def _sc_mesh():
    return plsc.VectorSubcoreMesh(core_axis_name="c", subcore_axis_name="s")


def _zero_offsets(D):
    offs = list(range(0, D - D % 16, 16))
    if D % 16:
        offs.append(D - 16)
    return offs


def _deg_body(dst_hbm, ew_hbm, out_hbm, dst_v, ew_v, zbuf, acc):
    cid = lax.axis_index("c")
    sid = lax.axis_index("s")
    wid = sid * NC + cid

    def zb(i, _):
        zbuf[pl.ds(i * 16, 16)] = jnp.zeros((16,), jnp.float32)
        return 0

    lax.fori_loop(0, TAIL // 16, zb, 0)

    @pl.when(sid < NS - 1)
    def _():
        pltpu.sync_copy(zbuf.at[pl.ds(0, CHUNK)], acc.at[pl.ds(sid * CHUNK, CHUNK)])

    @pl.when(sid == NS - 1)
    def _():
        pltpu.sync_copy(zbuf, acc.at[pl.ds((NS - 1) * CHUNK, TAIL)])

    plsc.subcore_barrier()

    pltpu.sync_copy(dst_hbm.at[wid], dst_v)
    pltpu.sync_copy(ew_hbm.at[wid], ew_v)

    def bb(j, _):
        pltpu.sync_copy(ew_v.at[j], acc.at[dst_v.at[j]], add=True)
        return 0

    lax.fori_loop(0, EK, bb, 0)
    plsc.subcore_barrier()

    @pl.when(sid < NS - 1)
    def _():
        pltpu.sync_copy(acc.at[pl.ds(sid * CHUNK, CHUNK)], zbuf.at[pl.ds(0, CHUNK)])
        pltpu.sync_copy(zbuf.at[pl.ds(0, CHUNK)],
                        out_hbm.at[pl.ds(cid * N_NODES + sid * CHUNK, CHUNK)])

    @pl.when(sid == NS - 1)
    def _():
        pltpu.sync_copy(acc.at[pl.ds((NS - 1) * CHUNK, TAIL)], zbuf)
        pltpu.sync_copy(zbuf,
                        out_hbm.at[pl.ds(cid * N_NODES + (NS - 1) * CHUNK, TAIL)])


def _sc_degree(dst3, ew3):
    kern = pl.kernel(
        _deg_body,
        out_type=jax.ShapeDtypeStruct((NC * N_NODES,), jnp.float32),
        mesh=_sc_mesh(),
        compiler_params=pltpu.CompilerParams(use_tc_tiling_on_sc=False),
        scratch_types=[
            pltpu.VMEM((EK, EB), jnp.int32),
            pltpu.VMEM((EK, EB), jnp.float32),
            pltpu.VMEM((TAIL,), jnp.float32),
            pltpu.VMEM_SHARED((N_NODES,), jnp.float32),
        ],
        name="sc_degree",
    )
    return kern(dst3, ew3)


def _copy_tile_range(sid, copy_chunk, copy_rem):
    for r in range(7):
        copy_chunk(r * EB)

    @pl.when(sid < NS - 1)
    def _():
        copy_rem(7 * EB, 64)

    @pl.when(sid == NS - 1)
    def _():
        copy_rem(7 * EB, EB)


def _edge_body(D, DR, h_hbm, src_hbm, dst_hbm, ew_hbm, out_hbm,
               src_v, dst_v, ew0, ew1, ew2, rows0, rows1, rows2,
               table, acc, gsem, ssem, esem):
    cid = lax.axis_index("c")
    sid = lax.axis_index("s")
    wid = sid * NC + cid
    tb = sid * CHUNK

    zoffs = _zero_offsets(D)

    def zrow(i, _):
        for c in zoffs:
            rows0[i, pl.ds(c, 16)] = jnp.zeros((16,), jnp.float32)
        return 0

    lax.fori_loop(0, EB, zrow, 0)

    def zero_chunk(off):
        pltpu.sync_copy(rows0, acc.at[pl.ds(tb + off, EB)])

    def zero_rem(off, n):
        pltpu.sync_copy(rows0.at[pl.ds(0, n)], acc.at[pl.ds(tb + off, n)])

    _copy_tile_range(sid, zero_chunk, zero_rem)

    def load_chunk(off):
        pltpu.sync_copy(h_hbm.at[pl.ds(tb + off, EB)], rows1)
        pltpu.sync_copy(rows1, table.at[pl.ds(tb + off, EB)])

    def load_rem(off, n):
        pltpu.sync_copy(h_hbm.at[pl.ds(tb + off, n)], rows1.at[pl.ds(0, n)])
        pltpu.sync_copy(rows1.at[pl.ds(0, n)], table.at[pl.ds(tb + off, n)])

    _copy_tile_range(sid, load_chunk, load_rem)

    plsc.subcore_barrier()

    pltpu.sync_copy(src_hbm.at[wid], src_v)
    pltpu.sync_copy(dst_hbm.at[wid], dst_v)

    ebufs = (ew0, ew1, ew2)
    rbufs = (rows0, rows1, rows2)

    def start_gather(j, buf, ebuf):
        pltpu.async_copy(table.at[src_v.at[j]], buf, gsem)
        pltpu.async_copy(ew_hbm.at[wid, j], ebuf, esem)

    def wait_gather(j, buf, ebuf):
        pltpu.make_async_copy(table.at[src_v.at[j]], buf, gsem).wait()
        pltpu.make_async_copy(ew_hbm.at[wid, j], ebuf, esem).wait()

    def start_scatter(j, buf):
        pltpu.async_copy(buf, acc.at[dst_v.at[j]], ssem, add=True)

    def wait_scatter(j, buf):
        pltpu.make_async_copy(buf, acc.at[dst_v.at[j]], ssem).wait()

    def scale(buf, ebuf):
        def rb(g, _):
            ewv = ebuf[pl.ds(g * 16, 16)]
            for l in range(16):
                wv = jnp.full((16,), ewv[l], jnp.float32)
                i = g * 16 + l
                for c in range(DR // 16):
                    buf[i, pl.ds(c * 16, 16)] = buf[i, pl.ds(c * 16, 16)] * wv
            return 0

        lax.fori_loop(0, EB // 16, rb, 0)

    start_gather(0, rows0, ew0)
    start_gather(1, rows1, ew1)

    def stepd(j, b, do_wait, do_gather):
        wait_gather(j, rbufs[b], ebufs[b])
        scale(rbufs[b], ebufs[b])
        start_scatter(j, rbufs[b])

        @pl.when(do_wait)
        def _():
            wait_scatter(j - 1, rbufs[(b + 2) % 3])

        @pl.when(do_gather)
        def _():
            start_gather(j + 2, rbufs[(b + 2) % 3], ebufs[(b + 2) % 3])

    @pl.loop(0, EK // 3)
    def _(p):
        j0 = 3 * p
        stepd(j0, 0, j0 >= 1, j0 + 2 < EK)
        stepd(j0 + 1, 1, True, j0 + 3 < EK)
        stepd(j0 + 2, 2, True, j0 + 4 < EK)

    stepd(EK - 2, 0, True, False)
    stepd(EK - 1, 1, True, False)
    wait_scatter(EK - 1, rows1)
    plsc.subcore_barrier()

    def out_chunk(off):
        pltpu.sync_copy(acc.at[pl.ds(tb + off, EB)], rows0)
        pltpu.sync_copy(rows0, out_hbm.at[cid, pl.ds(tb + off, EB)])

    def out_rem(off, n):
        pltpu.sync_copy(acc.at[pl.ds(tb + off, n)], rows0.at[pl.ds(0, n)])
        pltpu.sync_copy(rows0.at[pl.ds(0, n)], out_hbm.at[cid, pl.ds(tb + off, n)])

    _copy_tile_range(sid, out_chunk, out_rem)


def _sc_edge_agg(h, src3, dst3, ew3, DR):
    D = h.shape[1]
    kern = pl.kernel(
        functools.partial(_edge_body, D, DR),
        out_type=jax.ShapeDtypeStruct((NC, N_NODES, D), jnp.float32),
        mesh=_sc_mesh(),
        compiler_params=pltpu.CompilerParams(use_tc_tiling_on_sc=False),
        scratch_types=[
            pltpu.VMEM((EK, EB), jnp.int32),
            pltpu.VMEM((EK, EB), jnp.int32),
            pltpu.VMEM((EB,), jnp.float32),
            pltpu.VMEM((EB,), jnp.float32),
            pltpu.VMEM((EB,), jnp.float32),
            pltpu.VMEM((EB, D), jnp.float32),
            pltpu.VMEM((EB, D), jnp.float32),
            pltpu.VMEM((EB, D), jnp.float32),
            pltpu.VMEM_SHARED((N_NODES, D), jnp.float32),
            pltpu.VMEM_SHARED((N_NODES, D), jnp.float32),
            pltpu.SemaphoreType.DMA,
            pltpu.SemaphoreType.DMA,
            pltpu.SemaphoreType.DMA,
        ],
        name=f"sc_edge_agg_{D}",
    )
    return kern(h, src3, dst3, ew3)


BN = 2000


def _dinv_block(d0_ref, d1_ref):
    deg = 1.0 + d0_ref[...] + d1_ref[...]
    return jnp.where(deg > 0, lax.rsqrt(jnp.maximum(deg, 1e-12)), 0.0)


def _tc1_body(x_ref, w_ref, d0_ref, d1_ref, o_ref):
    dinv = _dinv_block(d0_ref, d1_ref)
    h = jnp.dot(x_ref[...], w_ref[...], preferred_element_type=jnp.float32,
                precision=lax.Precision.HIGHEST)
    o_ref[...] = h * dinv


def _tc1(x, W1p, d0, d1):
    grid = (N_NODES // BN,)
    return pl.pallas_call(
        _tc1_body,
        grid=grid,
        in_specs=[
            pl.BlockSpec((BN, D_FEAT), lambda i: (i, 0)),
            pl.BlockSpec((D_FEAT, DP1), lambda i: (0, 0)),
            pl.BlockSpec((BN, 1), lambda i: (i, 0)),
            pl.BlockSpec((BN, 1), lambda i: (i, 0)),
        ],
        out_specs=pl.BlockSpec((BN, DP1), lambda i: (i, 0)),
        out_shape=jax.ShapeDtypeStruct((N_NODES, DP1), jnp.float32),
    )(x, W1p, d0, d1)


def _tc2_body(a0_ref, a1_ref, h1d_ref, d0_ref, d1_ref, b1_ref, w2_ref, o_ref):
    dinv = _dinv_block(d0_ref, d1_ref)
    z = (a0_ref[...] + a1_ref[...] + h1d_ref[...]) * dinv + b1_ref[...]
    z = jnp.maximum(z, 0.0)
    h2 = jnp.dot(z, w2_ref[...], preferred_element_type=jnp.float32,
                 precision=lax.Precision.HIGHEST)
    o_ref[...] = h2 * dinv


def _tc2(a0, a1, h1d, d0, d1, b1p, W2p):
    grid = (N_NODES // BN,)
    return pl.pallas_call(
        _tc2_body,
        grid=grid,
        in_specs=[
            pl.BlockSpec((BN, DP1), lambda i: (i, 0)),
            pl.BlockSpec((BN, DP1), lambda i: (i, 0)),
            pl.BlockSpec((BN, DP1), lambda i: (i, 0)),
            pl.BlockSpec((BN, 1), lambda i: (i, 0)),
            pl.BlockSpec((BN, 1), lambda i: (i, 0)),
            pl.BlockSpec((1, DP1), lambda i: (0, 0)),
            pl.BlockSpec((DP1, DP2), lambda i: (0, 0)),
        ],
        out_specs=pl.BlockSpec((BN, DP2), lambda i: (i, 0)),
        out_shape=jax.ShapeDtypeStruct((N_NODES, DP2), jnp.float32),
    )(a0, a1, h1d, d0, d1, b1p, W2p)


def _tc3_body(a0_ref, a1_ref, h2d_ref, d0_ref, d1_ref, b2_ref, o_ref):
    dinv = _dinv_block(d0_ref, d1_ref)
    logits = (a0_ref[...] + a1_ref[...] + h2d_ref[...]) * dinv + b2_ref[...]
    li = lax.broadcasted_iota(jnp.int32, logits.shape, 1)
    lm = jnp.where(li < N_CLASSES, logits, jnp.float32(-1e30))
    m = jnp.max(lm, axis=1, keepdims=True)
    e = jnp.exp(lm - m)
    s = jnp.sum(e, axis=1, keepdims=True)
    o_ref[...] = logits - m - jnp.log(s)


def _tc3(a0, a1, h2d, d0, d1, b2p):
    grid = (N_NODES // BN,)
    return pl.pallas_call(
        _tc3_body,
        grid=grid,
        in_specs=[
            pl.BlockSpec((BN, DP2), lambda i: (i, 0)),
            pl.BlockSpec((BN, DP2), lambda i: (i, 0)),
            pl.BlockSpec((BN, DP2), lambda i: (i, 0)),
            pl.BlockSpec((BN, 1), lambda i: (i, 0)),
            pl.BlockSpec((BN, 1), lambda i: (i, 0)),
            pl.BlockSpec((1, DP2), lambda i: (0, 0)),
        ],
        out_specs=pl.BlockSpec((BN, DP2), lambda i: (i, 0)),
        out_shape=jax.ShapeDtypeStruct((N_NODES, DP2), jnp.float32),
    )(a0, a1, h2d, d0, d1, b2p)


def kernel(x, edge_index, edge_attr, W1, b1, W2, b2):
    src = edge_index[0].astype(jnp.int32)
    dst = edge_index[1].astype(jnp.int32)
    ew = edge_attr.astype(jnp.float32)

    src3 = src.reshape(NW, EK, EB)
    dst3 = dst.reshape(NW, EK, EB)
    ew3 = ew.reshape(NW, EK, EB)

    W1p = jnp.pad(W1.astype(jnp.float32), ((0, 0), (0, DP1 - HIDDEN)))
    b1p = jnp.pad(b1.astype(jnp.float32), (0, DP1 - HIDDEN)).reshape(1, DP1)
    W2p = jnp.pad(W2.astype(jnp.float32),
                  ((0, DP1 - HIDDEN), (0, DP2 - N_CLASSES)))
    b2p = jnp.pad(b2.astype(jnp.float32), (0, DP2 - N_CLASSES)).reshape(1, DP2)

    degp = _sc_degree(dst3, ew3).reshape(NC, N_NODES)
    d0 = degp[0].reshape(N_NODES, 1)
    d1 = degp[1].reshape(N_NODES, 1)

    h1d = _tc1(x, W1p, d0, d1)
    a1 = _sc_edge_agg(h1d, src3, dst3, ew3, HIDDEN)
    h2d = _tc2(a1[0], a1[1], h1d, d0, d1, b1p, W2p)
    a2 = _sc_edge_agg(h2d, src3, dst3, ew3, 48)
    out = _tc3(a2[0], a2[1], h2d, d0, d1, b2p)
    return out[:, :N_CLASSES]
